# baseline (speedup 1.0000x reference)
import jax
import jax.numpy as jnp
from jax.experimental import pallas as pl
from jax.experimental.pallas import tpu as pltpu

N = 32768
D = 256
H = 8
B = 16
BH = B * H
BLK = 8192


def _agg_kernel(seg_ref, att_ref, feat_ref, avg_ref, w_ref, sum_acc, w_acc):
    i = pl.program_id(0)
    nsteps = pl.num_programs(0)

    @pl.when(i == 0)
    def _init():
        sum_acc[...] = jnp.zeros_like(sum_acc)
        w_acc[...] = jnp.zeros_like(w_acc)

    seg = seg_ref[...].reshape(1, BLK)
    att_t = att_ref[...]
    feat = feat_ref[...]

    b_idx = jax.lax.broadcasted_iota(jnp.int32, (BH, BLK), 0) // H
    seg_b = jnp.broadcast_to(seg, (BH, BLK))
    att_rep = jnp.tile(att_t, (B, 1))
    pt = jnp.where(seg_b == b_idx, att_rep, 0.0)

    dn = (((1,), (0,)), ((), ()))
    sum_acc[...] += jax.lax.dot_general(
        pt, feat, dn, preferred_element_type=jnp.float32
    )
    w_acc[...] += jax.lax.dot_general(
        pt, jnp.ones((BLK, 1), jnp.float32), dn,
        preferred_element_type=jnp.float32,
    )

    @pl.when(i == nsteps - 1)
    def _finalize():
        w = w_acc[...]
        safe = jnp.where(w == 0.0, 1.0, w)
        avg = jnp.where(w == 0.0, 0.0, sum_acc[...] / safe)
        avg = jnp.where(jnp.isnan(avg), 1e-05, avg)
        avg_ref[...] = avg.reshape(B, H, D)
        w_ref[...] = w.reshape(B, H)


def kernel(flat_features, flat_att, segment_ids):
    seg2d = segment_ids.reshape(N // 128, 128)
    att_t = flat_att.T
    grid = N // BLK
    avg, w = pl.pallas_call(
        _agg_kernel,
        grid=(grid,),
        in_specs=[
            pl.BlockSpec((BLK // 128, 128), lambda i: (i, 0)),
            pl.BlockSpec((H, BLK), lambda i: (0, i)),
            pl.BlockSpec((BLK, D), lambda i: (i, 0)),
        ],
        out_specs=[
            pl.BlockSpec((B, H, D), lambda i: (0, 0, 0)),
            pl.BlockSpec((B, H), lambda i: (0, 0)),
        ],
        out_shape=[
            jax.ShapeDtypeStruct((B, H, D), jnp.float32),
            jax.ShapeDtypeStruct((B, H), jnp.float32),
        ],
        scratch_shapes=[
            pltpu.VMEM((BH, D), jnp.float32),
            pltpu.VMEM((BH, 1), jnp.float32),
        ],
    )(seg2d, att_t, flat_features)
    return avg, w


# speedup vs baseline: 1.0977x; 1.0977x over previous
import jax
import jax.numpy as jnp
from jax.experimental import pallas as pl
from jax.experimental.pallas import tpu as pltpu

N = 32768
D = 256
H = 8
B = 16
BH = B * H
BLK = 8192


def _agg_kernel(seg_ref, att_ref, feat_ref, avg_ref, w_ref, sum_acc, w_acc):
    i = pl.program_id(0)
    nsteps = pl.num_programs(0)

    @pl.when(i == 0)
    def _init():
        sum_acc[...] = jnp.zeros_like(sum_acc)
        w_acc[...] = jnp.zeros_like(w_acc)

    seg = seg_ref[...].reshape(1, BLK)
    att_t = att_ref[...]
    feat = feat_ref[...]

    b_idx = jax.lax.broadcasted_iota(jnp.int32, (BH, BLK), 0) // H
    seg_b = jnp.broadcast_to(seg, (BH, BLK))
    att_rep = jnp.tile(att_t, (B, 1))
    pt = jnp.where(seg_b == b_idx, att_rep, 0.0)

    dn = (((1,), (0,)), ((), ()))
    sum_acc[...] += jax.lax.dot_general(
        pt, feat, dn, preferred_element_type=jnp.float32
    )
    w_acc[...] += jax.lax.dot_general(
        pt, jnp.ones((BLK, 1), jnp.float32), dn,
        preferred_element_type=jnp.float32,
    )

    @pl.when(i == nsteps - 1)
    def _finalize():
        w = w_acc[...]
        safe = jnp.where(w == 0.0, 1.0, w)
        avg = jnp.where(w == 0.0, 0.0, sum_acc[...] / safe)
        avg = jnp.where(jnp.isnan(avg), 1e-05, avg)
        avg_ref[...] = avg.reshape(B, H, D)
        w_ref[...] = w.reshape(B, H).T


def kernel(flat_features, flat_att, segment_ids):
    seg2d = segment_ids.reshape(N // 128, 128)
    att_t = flat_att.T
    grid = N // BLK
    avg, w = pl.pallas_call(
        _agg_kernel,
        grid=(grid,),
        in_specs=[
            pl.BlockSpec((BLK // 128, 128), lambda i: (i, 0)),
            pl.BlockSpec((H, BLK), lambda i: (0, i)),
            pl.BlockSpec((BLK, D), lambda i: (i, 0)),
        ],
        out_specs=[
            pl.BlockSpec((B, H, D), lambda i: (0, 0, 0)),
            pl.BlockSpec((H, B), lambda i: (0, 0)),
        ],
        out_shape=[
            jax.ShapeDtypeStruct((B, H, D), jnp.float32),
            jax.ShapeDtypeStruct((H, B), jnp.float32),
        ],
        scratch_shapes=[
            pltpu.VMEM((BH, D), jnp.float32),
            pltpu.VMEM((BH, 1), jnp.float32),
        ],
    )(seg2d, att_t, flat_features)
    return avg, w.T
